# Initial kernel scaffold; baseline (speedup 1.0000x reference)
import jax
import jax.numpy as jnp
from jax.experimental import pallas as pl


def kernel(x, edge_index, W1, b1, W2, b2, Wa, ba):
    raise NotImplementedError("write your pallas kernel here")


# speedup vs baseline: 50.0075x; 50.0075x over previous
import functools

import jax
import jax.numpy as jnp
from jax import lax
from jax.experimental import pallas as pl
from jax.experimental.pallas import tpu as pltpu
from jax.experimental.pallas import tpu_sc as plsc

N = 100000
NPAD = 100352
SLICE = NPAD // 16
E = 3200000
EPAD = 3276800
NB = EPAD // 128
MB_W = NB // 32
KC = 16
OUTER = MB_W // KC
D = 8
H = 16
F32 = jnp.float32

_mesh = plsc.VectorSubcoreMesh(core_axis_name="c", subcore_axis_name="s")
_sc_params = pltpu.CompilerParams(use_tc_tiling_on_sc=False)


def _deg_body(edge_ref, ones_hbm, zs_hbm, deg_out, dst_buf, ones_vm, deg_sh,
              sem):
    c = lax.axis_index("c")
    s = lax.axis_index("s")
    wid = c * 16 + s
    pltpu.sync_copy(ones_hbm, ones_vm)
    pltpu.sync_copy(zs_hbm, deg_sh.at[pl.ds(s * SLICE, SLICE)])
    plsc.subcore_barrier()

    def outer(i, carry):
        boff = wid * MB_W + i * KC
        pltpu.sync_copy(edge_ref.at[1, pl.ds(boff, KC)], dst_buf)

        def inner(j, carry2):
            pltpu.sync_copy(ones_vm, deg_sh.at[dst_buf.at[j]], add=True)
            return carry2

        return lax.fori_loop(0, KC, inner, carry)

    lax.fori_loop(0, OUTER, outer, 0)
    plsc.subcore_barrier()
    pltpu.sync_copy(deg_sh.at[pl.ds(s * SLICE, SLICE)],
                    deg_out.at[c, pl.ds(s * SLICE, SLICE)])


_deg_call = pl.kernel(
    _deg_body,
    out_type=jax.ShapeDtypeStruct((2, NPAD), F32),
    mesh=_mesh,
    compiler_params=_sc_params,
    scratch_types=[
        pltpu.VMEM((KC, 128), jnp.int32),
        pltpu.VMEM((128,), F32),
        pltpu.VMEM_SHARED((NPAD,), F32),
        pltpu.SemaphoreType.DMA,
    ],
)


def _main_body(edge_ref, xn_hbm, norm_hbm, zr_hbm, zs_hbm,
               accx_out, s_out,
               src_buf, dst_buf, rows_v, nd_v, accx_sh, s_sh, sem, sem2):
    c = lax.axis_index("c")
    s = lax.axis_index("s")
    wid = c * 16 + s
    pltpu.sync_copy(zr_hbm, accx_sh.at[pl.ds(s * SLICE, SLICE)])
    pltpu.sync_copy(zs_hbm, s_sh.at[pl.ds(s * SLICE, SLICE)])
    plsc.subcore_barrier()

    def outer(i, carry):
        boff = wid * MB_W + i * KC
        pltpu.sync_copy(edge_ref.at[0, pl.ds(boff, KC)], src_buf)
        pltpu.sync_copy(edge_ref.at[1, pl.ds(boff, KC)], dst_buf)

        def inner(j, carry2):
            cp1 = pltpu.async_copy(xn_hbm.at[src_buf.at[j]], rows_v, sem)
            cp2 = pltpu.async_copy(norm_hbm.at[dst_buf.at[j]], nd_v, sem2)
            cp1.wait()
            cp2.wait()
            pltpu.sync_copy(rows_v, accx_sh.at[dst_buf.at[j]], add=True)
            pltpu.sync_copy(nd_v, s_sh.at[src_buf.at[j]], add=True)
            return carry2

        return lax.fori_loop(0, KC, inner, carry)

    lax.fori_loop(0, OUTER, outer, 0)
    plsc.subcore_barrier()
    pltpu.sync_copy(accx_sh.at[pl.ds(s * SLICE, SLICE)],
                    accx_out.at[c, pl.ds(s * SLICE, SLICE)])
    pltpu.sync_copy(s_sh.at[pl.ds(s * SLICE, SLICE)],
                    s_out.at[c, pl.ds(s * SLICE, SLICE)])


_main_call = pl.kernel(
    _main_body,
    out_type=(
        jax.ShapeDtypeStruct((2, NPAD, D), F32),
        jax.ShapeDtypeStruct((2, NPAD), F32),
    ),
    mesh=_mesh,
    compiler_params=_sc_params,
    scratch_types=[
        pltpu.VMEM((KC, 128), jnp.int32),
        pltpu.VMEM((KC, 128), jnp.int32),
        pltpu.VMEM((128, D), F32),
        pltpu.VMEM((128,), F32),
        pltpu.VMEM_SHARED((NPAD, D), F32),
        pltpu.VMEM_SHARED((NPAD,), F32),
        pltpu.SemaphoreType.DMA,
        pltpu.SemaphoreType.DMA,
    ],
)


def _tc1_body(d0_ref, d1_ref, x_ref, norm_ref, xn_ref):
    deg = d0_ref[...] + d1_ref[...] + 1.0
    nrm = lax.rsqrt(deg)
    norm_ref[...] = nrm
    xn_ref[...] = x_ref[...] * nrm


_GRID1 = 16


def _tc1(d0c, d1c, x_pad):
    blk = NPAD // _GRID1
    return pl.pallas_call(
        _tc1_body,
        grid=(_GRID1,),
        in_specs=[
            pl.BlockSpec((blk, 1), lambda i: (i, 0)),
            pl.BlockSpec((blk, 1), lambda i: (i, 0)),
            pl.BlockSpec((blk, D), lambda i: (i, 0)),
        ],
        out_specs=[
            pl.BlockSpec((blk, 1), lambda i: (i, 0)),
            pl.BlockSpec((blk, D), lambda i: (i, 0)),
        ],
        out_shape=[
            jax.ShapeDtypeStruct((NPAD, 1), F32),
            jax.ShapeDtypeStruct((NPAD, D), F32),
        ],
    )(d0c, d1c, x_pad)


def _tc2_body(a0_ref, a1_ref, xn_ref, s0_ref, s1_ref, norm_ref,
              w1_ref, b1_ref, w2_ref, b2_ref, wa_ref, ba_ref,
              out_ref, r_acc):
    k = pl.program_id(0)

    @pl.when(k == 0)
    def _():
        r_acc[...] = jnp.zeros_like(r_acc)

    blk = a0_ref.shape[0]
    nrm = norm_ref[...]
    u = a0_ref[...] + a1_ref[...] + xn_ref[...]
    h1 = jax.nn.relu(nrm * jnp.dot(u, w1_ref[...],
                                   preferred_element_type=F32) + b1_ref[...])
    wv = nrm * (nrm + s0_ref[...] + s1_ref[...])
    gidx = lax.broadcasted_iota(jnp.int32, (blk, 1), 0) + k * blk
    wv = jnp.where(gidx < N, wv, 0.0)
    r_acc[...] += jnp.sum(wv * h1, axis=0, keepdims=True)

    @pl.when(k == _GRID1 - 1)
    def _():
        feat = jnp.dot(r_acc[...], w2_ref[...],
                       preferred_element_type=F32) * (1.0 / N) + b2_ref[...]
        out_ref[...] = jnp.dot(feat, wa_ref[...],
                               preferred_element_type=F32) + ba_ref[...]


def _tc2(a0, a1, xn, s0c, s1c, norm_c, W1p, b1r, W2, b2r, Wa, bar):
    blk = NPAD // _GRID1
    wspec = lambda shape: pl.BlockSpec(shape, lambda i: (0, 0))
    return pl.pallas_call(
        _tc2_body,
        grid=(_GRID1,),
        in_specs=[
            pl.BlockSpec((blk, D), lambda i: (i, 0)),
            pl.BlockSpec((blk, D), lambda i: (i, 0)),
            pl.BlockSpec((blk, D), lambda i: (i, 0)),
            pl.BlockSpec((blk, 1), lambda i: (i, 0)),
            pl.BlockSpec((blk, 1), lambda i: (i, 0)),
            pl.BlockSpec((blk, 1), lambda i: (i, 0)),
            wspec((D, H)),
            wspec((1, H)),
            wspec((H, 64)),
            wspec((1, 64)),
            wspec((64, 10)),
            wspec((1, 10)),
        ],
        out_specs=pl.BlockSpec((1, 10), lambda i: (0, 0)),
        out_shape=jax.ShapeDtypeStruct((1, 10), F32),
        scratch_shapes=[pltpu.VMEM((1, H), F32)],
    )(a0, a1, xn, s0c, s1c, norm_c, W1p, b1r, W2, b2r, Wa, bar)


def kernel(x, edge_index, W1, b1, W2, b2, Wa, ba):
    pad_e = jnp.full((2, EPAD - E), N, dtype=edge_index.dtype)
    edge_r = jnp.concatenate([edge_index, pad_e], axis=1).reshape(2, NB, 128)
    x_pad = jnp.zeros((NPAD, D), F32).at[:N, :5].set(x)
    W1p = jnp.zeros((D, H), F32).at[:5, :].set(W1)
    ones128 = jnp.ones((128,), F32)
    zeros_s = jnp.zeros((SLICE,), F32)
    zeros_r = jnp.zeros((SLICE, D), F32)

    deg_out = _deg_call(edge_r, ones128, zeros_s)

    d0c = deg_out[0].reshape(NPAD, 1)
    d1c = deg_out[1].reshape(NPAD, 1)
    norm_c, xn = _tc1(d0c, d1c, x_pad)

    accx_out, s_out = _main_call(edge_r, xn, norm_c.reshape(NPAD), zeros_r,
                                 zeros_s)

    logits = _tc2(accx_out[0], accx_out[1], xn,
                  s_out[0].reshape(NPAD, 1), s_out[1].reshape(NPAD, 1),
                  norm_c, W1p, b1.reshape(1, H), W2, b2.reshape(1, 64),
                  Wa, ba.reshape(1, 10))
    return logits


# speedup vs baseline: 74.1329x; 1.4824x over previous
import jax
import jax.numpy as jnp
from jax import lax
from jax.experimental import pallas as pl
from jax.experimental.pallas import tpu as pltpu
from jax.experimental.pallas import tpu_sc as plsc

N = 100000
NPAD = 100352
SLICE = NPAD // 16
E = 3200000
NB = E // 128
MB_LO = NB // 32
MB_EX = NB % 32
NG_MAX = MB_LO + 1
D = 8
H = 16
P = 4
F32 = jnp.float32

_mesh = plsc.VectorSubcoreMesh(core_axis_name="c", subcore_axis_name="s")
_sc_params = pltpu.CompilerParams(use_tc_tiling_on_sc=False)


def _worker_range(c, s):
    wid = c * 16 + s
    start = wid * MB_LO + jnp.minimum(wid, MB_EX)
    ng = MB_LO + jnp.where(wid < MB_EX, 1, 0)
    return start, ng


def _deg_body(edge_ref, ones_hbm, zs_hbm, deg_out,
              dst_buf, ones_vm, deg_sh, sem_e, sem_s):
    c = lax.axis_index("c")
    s = lax.axis_index("s")
    start, ng = _worker_range(c, s)
    pltpu.sync_copy(ones_hbm, ones_vm)
    pltpu.sync_copy(zs_hbm, deg_sh.at[pl.ds(s * SLICE, SLICE)])
    plsc.subcore_barrier()

    def body(t, carry):
        sl = lax.rem(t, P)

        @pl.when((t >= P) & (t - P < ng))
        def _():
            pltpu.make_async_copy(ones_vm, deg_sh.at[dst_buf.at[sl]],
                                  sem_s.at[sl]).wait()

        @pl.when(t < ng)
        def _():
            pltpu.async_copy(edge_ref.at[1, start + t], dst_buf.at[sl],
                             sem_e.at[sl])

        @pl.when((t >= 1) & (t - 1 < ng))
        def _():
            sg = lax.rem(t - 1, P)
            pltpu.make_async_copy(edge_ref.at[1, start + t - 1],
                                  dst_buf.at[sg], sem_e.at[sg]).wait()
            pltpu.async_copy(ones_vm, deg_sh.at[dst_buf.at[sg]],
                             sem_s.at[sg], add=True)

        return carry

    lax.fori_loop(0, NG_MAX + P, body, 0)
    plsc.subcore_barrier()
    pltpu.sync_copy(deg_sh.at[pl.ds(s * SLICE, SLICE)],
                    deg_out.at[c, pl.ds(s * SLICE, SLICE)])


_deg_call = pl.kernel(
    _deg_body,
    out_type=jax.ShapeDtypeStruct((2, NPAD), F32),
    mesh=_mesh,
    compiler_params=_sc_params,
    scratch_types=[
        pltpu.VMEM((P, 128), jnp.int32),
        pltpu.VMEM((128,), F32),
        pltpu.VMEM_SHARED((NPAD,), F32),
        pltpu.SemaphoreType.DMA((P,)),
        pltpu.SemaphoreType.DMA((P,)),
    ],
)


def _main_body(edge_ref, xn_hbm, norm_hbm, zr_hbm, zs_hbm,
               accx_out, s_out,
               src_buf, dst_buf, rows_v, nd_v, accx_sh, s_sh,
               sem_e, sem_g, sem_s):
    c = lax.axis_index("c")
    s = lax.axis_index("s")
    start, ng = _worker_range(c, s)
    pltpu.sync_copy(zr_hbm, accx_sh.at[pl.ds(s * SLICE, SLICE)])
    pltpu.sync_copy(zs_hbm, s_sh.at[pl.ds(s * SLICE, SLICE)])
    plsc.subcore_barrier()

    def body(t, carry):
        sl = lax.rem(t, P)

        @pl.when((t >= P) & (t - P < ng))
        def _():
            pltpu.make_async_copy(rows_v.at[sl],
                                  accx_sh.at[dst_buf.at[sl]],
                                  sem_s.at[sl]).wait()
            pltpu.make_async_copy(nd_v.at[sl], s_sh.at[src_buf.at[sl]],
                                  sem_s.at[sl]).wait()

        @pl.when(t < ng)
        def _():
            pltpu.async_copy(edge_ref.at[0, start + t], src_buf.at[sl],
                             sem_e.at[sl])
            pltpu.async_copy(edge_ref.at[1, start + t], dst_buf.at[sl],
                             sem_e.at[sl])

        @pl.when((t >= 1) & (t - 1 < ng))
        def _():
            sg = lax.rem(t - 1, P)
            pltpu.make_async_copy(edge_ref.at[0, start + t - 1],
                                  src_buf.at[sg], sem_e.at[sg]).wait()
            pltpu.make_async_copy(edge_ref.at[1, start + t - 1],
                                  dst_buf.at[sg], sem_e.at[sg]).wait()
            pltpu.async_copy(xn_hbm.at[src_buf.at[sg]], rows_v.at[sg],
                             sem_g.at[sg])
            pltpu.async_copy(norm_hbm.at[dst_buf.at[sg]], nd_v.at[sg],
                             sem_g.at[sg])

        @pl.when((t >= 2) & (t - 2 < ng))
        def _():
            ss = lax.rem(t - 2, P)
            pltpu.make_async_copy(xn_hbm.at[src_buf.at[ss]], rows_v.at[ss],
                                  sem_g.at[ss]).wait()
            pltpu.make_async_copy(norm_hbm.at[dst_buf.at[ss]], nd_v.at[ss],
                                  sem_g.at[ss]).wait()
            pltpu.async_copy(rows_v.at[ss], accx_sh.at[dst_buf.at[ss]],
                             sem_s.at[ss], add=True)
            pltpu.async_copy(nd_v.at[ss], s_sh.at[src_buf.at[ss]],
                             sem_s.at[ss], add=True)

        return carry

    lax.fori_loop(0, NG_MAX + P, body, 0)
    plsc.subcore_barrier()
    pltpu.sync_copy(accx_sh.at[pl.ds(s * SLICE, SLICE)],
                    accx_out.at[c, pl.ds(s * SLICE, SLICE)])
    pltpu.sync_copy(s_sh.at[pl.ds(s * SLICE, SLICE)],
                    s_out.at[c, pl.ds(s * SLICE, SLICE)])


_main_call = pl.kernel(
    _main_body,
    out_type=(
        jax.ShapeDtypeStruct((2, NPAD, D), F32),
        jax.ShapeDtypeStruct((2, NPAD), F32),
    ),
    mesh=_mesh,
    compiler_params=_sc_params,
    scratch_types=[
        pltpu.VMEM((P, 128), jnp.int32),
        pltpu.VMEM((P, 128), jnp.int32),
        pltpu.VMEM((P, 128, D), F32),
        pltpu.VMEM((P, 128), F32),
        pltpu.VMEM_SHARED((NPAD, D), F32),
        pltpu.VMEM_SHARED((NPAD,), F32),
        pltpu.SemaphoreType.DMA((P,)),
        pltpu.SemaphoreType.DMA((P,)),
        pltpu.SemaphoreType.DMA((P,)),
    ],
)


def _tc1_body(d0_ref, d1_ref, x_ref, norm_ref, xn_ref):
    deg = d0_ref[...] + d1_ref[...] + 1.0
    nrm = lax.rsqrt(deg)
    norm_ref[...] = nrm
    xn_ref[...] = x_ref[...] * nrm


_GRID1 = 16


def _tc1(d0c, d1c, x_pad):
    blk = NPAD // _GRID1
    return pl.pallas_call(
        _tc1_body,
        grid=(_GRID1,),
        in_specs=[
            pl.BlockSpec((blk, 1), lambda i: (i, 0)),
            pl.BlockSpec((blk, 1), lambda i: (i, 0)),
            pl.BlockSpec((blk, D), lambda i: (i, 0)),
        ],
        out_specs=[
            pl.BlockSpec((blk, 1), lambda i: (i, 0)),
            pl.BlockSpec((blk, D), lambda i: (i, 0)),
        ],
        out_shape=[
            jax.ShapeDtypeStruct((NPAD, 1), F32),
            jax.ShapeDtypeStruct((NPAD, D), F32),
        ],
    )(d0c, d1c, x_pad)


def _tc2_body(a0_ref, a1_ref, xn_ref, s0_ref, s1_ref, norm_ref,
              w1_ref, b1_ref, w2_ref, b2_ref, wa_ref, ba_ref,
              out_ref, r_acc):
    k = pl.program_id(0)

    @pl.when(k == 0)
    def _():
        r_acc[...] = jnp.zeros_like(r_acc)

    blk = a0_ref.shape[0]
    nrm = norm_ref[...]
    u = a0_ref[...] + a1_ref[...] + xn_ref[...]
    h1 = jax.nn.relu(nrm * jnp.dot(u, w1_ref[...],
                                   preferred_element_type=F32) + b1_ref[...])
    wv = nrm * (nrm + s0_ref[...] + s1_ref[...])
    gidx = lax.broadcasted_iota(jnp.int32, (blk, 1), 0) + k * blk
    wv = jnp.where(gidx < N, wv, 0.0)
    r_acc[...] += jnp.sum(wv * h1, axis=0, keepdims=True)

    @pl.when(k == _GRID1 - 1)
    def _():
        feat = jnp.dot(r_acc[...], w2_ref[...],
                       preferred_element_type=F32) * (1.0 / N) + b2_ref[...]
        out_ref[...] = jnp.dot(feat, wa_ref[...],
                               preferred_element_type=F32) + ba_ref[...]


def _tc2(a0, a1, xn, s0c, s1c, norm_c, W1p, b1r, W2, b2r, Wa, bar):
    blk = NPAD // _GRID1
    wspec = lambda shape: pl.BlockSpec(shape, lambda i: (0, 0))
    return pl.pallas_call(
        _tc2_body,
        grid=(_GRID1,),
        in_specs=[
            pl.BlockSpec((blk, D), lambda i: (i, 0)),
            pl.BlockSpec((blk, D), lambda i: (i, 0)),
            pl.BlockSpec((blk, D), lambda i: (i, 0)),
            pl.BlockSpec((blk, 1), lambda i: (i, 0)),
            pl.BlockSpec((blk, 1), lambda i: (i, 0)),
            pl.BlockSpec((blk, 1), lambda i: (i, 0)),
            wspec((D, H)),
            wspec((1, H)),
            wspec((H, 64)),
            wspec((1, 64)),
            wspec((64, 10)),
            wspec((1, 10)),
        ],
        out_specs=pl.BlockSpec((1, 10), lambda i: (0, 0)),
        out_shape=jax.ShapeDtypeStruct((1, 10), F32),
        scratch_shapes=[pltpu.VMEM((1, H), F32)],
    )(a0, a1, xn, s0c, s1c, norm_c, W1p, b1r, W2, b2r, Wa, bar)


def kernel(x, edge_index, W1, b1, W2, b2, Wa, ba):
    edge_r = edge_index.reshape(2, NB, 128)
    x_pad = jnp.zeros((NPAD, D), F32).at[:N, :5].set(x)
    W1p = jnp.zeros((D, H), F32).at[:5, :].set(W1)
    ones128 = jnp.ones((128,), F32)
    zeros_s = jnp.zeros((SLICE,), F32)
    zeros_r = jnp.zeros((SLICE, D), F32)

    deg_out = _deg_call(edge_r, ones128, zeros_s)

    d0c = deg_out[0].reshape(NPAD, 1)
    d1c = deg_out[1].reshape(NPAD, 1)
    norm_c, xn = _tc1(d0c, d1c, x_pad)

    accx_out, s_out = _main_call(edge_r, xn, norm_c.reshape(NPAD), zeros_r,
                                 zeros_s)

    logits = _tc2(accx_out[0], accx_out[1], xn,
                  s_out[0].reshape(NPAD, 1), s_out[1].reshape(NPAD, 1),
                  norm_c, W1p, b1.reshape(1, H), W2, b2.reshape(1, 64),
                  Wa, ba.reshape(1, 10))
    return logits


# speedup vs baseline: 86.1240x; 1.1618x over previous
import jax
import jax.numpy as jnp
from jax import lax
from jax.experimental import pallas as pl
from jax.experimental.pallas import tpu as pltpu
from jax.experimental.pallas import tpu_sc as plsc

N = 100000
NPAD = 100352
SLICE = NPAD // 16
E = 3200000
NB = E // 128
KC = 8
NCH = NB // KC
CH_LO = NCH // 32
CH_EX = NCH % 32
NG_MAX = (CH_LO + 1) * KC
D = 8
H = 16
P = 4
F32 = jnp.float32

_mesh = plsc.VectorSubcoreMesh(core_axis_name="c", subcore_axis_name="s")
_sc_params = pltpu.CompilerParams(use_tc_tiling_on_sc=False)


def _worker_range(c, s):
    wid = c * 16 + s
    start = (wid * CH_LO + jnp.minimum(wid, CH_EX)) * KC
    ng = (CH_LO + jnp.where(wid < CH_EX, 1, 0)) * KC
    return start, ng


def _deg_body(edge_ref, ones_hbm, zs_hbm, deg_out,
              dst_buf, ones_vm, deg_sh, sem_e, sem_s):
    c = lax.axis_index("c")
    s = lax.axis_index("s")
    start, ng = _worker_range(c, s)
    pltpu.sync_copy(ones_hbm, ones_vm)
    pltpu.sync_copy(zs_hbm, deg_sh.at[pl.ds(s * SLICE, SLICE)])
    plsc.subcore_barrier()
    pltpu.async_copy(edge_ref.at[1, pl.ds(start, KC)], dst_buf.at[0],
                     sem_e.at[0])

    def body(t, carry):
        sl = lax.rem(lax.div(t, KC), 3)
        row = lax.rem(t, KC)

        @pl.when((row == 0) & (t + KC < ng))
        def _():
            so = lax.rem(lax.div(t, KC) + 1, 3)
            pltpu.async_copy(edge_ref.at[1, pl.ds(start + t + KC, KC)],
                             dst_buf.at[so], sem_e.at[so])

        @pl.when((row == 0) & (t < ng))
        def _():
            pltpu.make_async_copy(edge_ref.at[1, pl.ds(start + t, KC)],
                                  dst_buf.at[sl], sem_e.at[sl]).wait()

        @pl.when((t >= P) & (t - P < ng))
        def _():
            sp = lax.rem(t - P, P)
            sp_sl = lax.rem(lax.div(t - P, KC), 3)
            sp_row = lax.rem(t - P, KC)
            pltpu.make_async_copy(ones_vm,
                                  deg_sh.at[dst_buf.at[sp_sl, sp_row]],
                                  sem_s.at[sp]).wait()

        @pl.when(t < ng)
        def _():
            pltpu.async_copy(ones_vm, deg_sh.at[dst_buf.at[sl, row]],
                             sem_s.at[lax.rem(t, P)], add=True)

        return carry

    lax.fori_loop(0, NG_MAX + P, body, 0)
    plsc.subcore_barrier()
    pltpu.sync_copy(deg_sh.at[pl.ds(s * SLICE, SLICE)],
                    deg_out.at[c, pl.ds(s * SLICE, SLICE)])


_deg_call = pl.kernel(
    _deg_body,
    out_type=jax.ShapeDtypeStruct((2, NPAD), F32),
    mesh=_mesh,
    compiler_params=_sc_params,
    scratch_types=[
        pltpu.VMEM((3, KC, 128), jnp.int32),
        pltpu.VMEM((128,), F32),
        pltpu.VMEM_SHARED((NPAD,), F32),
        pltpu.SemaphoreType.DMA((3,)),
        pltpu.SemaphoreType.DMA((P,)),
    ],
)


def _main_body(edge_ref, xn_hbm, norm_hbm, zr_hbm, zs_hbm,
               accx_out, s_out,
               src_buf, dst_buf, rows_v, nd_v, accx_sh, s_sh,
               sem_e, sem_g, sem_s):
    c = lax.axis_index("c")
    s = lax.axis_index("s")
    start, ng = _worker_range(c, s)
    pltpu.sync_copy(zr_hbm, accx_sh.at[pl.ds(s * SLICE, SLICE)])
    pltpu.sync_copy(zs_hbm, s_sh.at[pl.ds(s * SLICE, SLICE)])
    plsc.subcore_barrier()
    pltpu.async_copy(edge_ref.at[0, pl.ds(start, KC)], src_buf.at[0],
                     sem_e.at[0])
    pltpu.async_copy(edge_ref.at[1, pl.ds(start, KC)], dst_buf.at[0],
                     sem_e.at[0])

    def body(t, carry):
        sl = lax.rem(lax.div(t, KC), 3)
        row = lax.rem(t, KC)

        @pl.when((row == 0) & (t + KC < ng))
        def _():
            so = lax.rem(lax.div(t, KC) + 1, 3)
            pltpu.async_copy(edge_ref.at[0, pl.ds(start + t + KC, KC)],
                             src_buf.at[so], sem_e.at[so])
            pltpu.async_copy(edge_ref.at[1, pl.ds(start + t + KC, KC)],
                             dst_buf.at[so], sem_e.at[so])

        @pl.when((row == 0) & (t < ng))
        def _():
            pltpu.make_async_copy(edge_ref.at[0, pl.ds(start + t, KC)],
                                  src_buf.at[sl], sem_e.at[sl]).wait()
            pltpu.make_async_copy(edge_ref.at[1, pl.ds(start + t, KC)],
                                  dst_buf.at[sl], sem_e.at[sl]).wait()

        @pl.when((t >= 2) & (t - 2 < ng))
        def _():
            g2 = t - 2
            ss = lax.rem(g2, P)
            e2 = lax.rem(lax.div(g2, KC), 3)
            r2 = lax.rem(g2, KC)
            pltpu.make_async_copy(xn_hbm.at[src_buf.at[e2, r2]],
                                  rows_v.at[ss], sem_g.at[ss]).wait()
            pltpu.make_async_copy(norm_hbm.at[dst_buf.at[e2, r2]],
                                  nd_v.at[ss], sem_g.at[ss]).wait()
            pltpu.async_copy(rows_v.at[ss], accx_sh.at[dst_buf.at[e2, r2]],
                             sem_s.at[ss], add=True)
            pltpu.async_copy(nd_v.at[ss], s_sh.at[src_buf.at[e2, r2]],
                             sem_s.at[ss], add=True)

        @pl.when(t < ng)
        def _():
            sg = lax.rem(t, P)

            @pl.when(t >= P)
            def _():
                gp = t - P
                ep = lax.rem(lax.div(gp, KC), 3)
                rp = lax.rem(gp, KC)
                pltpu.make_async_copy(rows_v.at[sg],
                                      accx_sh.at[dst_buf.at[ep, rp]],
                                      sem_s.at[sg]).wait()
                pltpu.make_async_copy(nd_v.at[sg],
                                      s_sh.at[src_buf.at[ep, rp]],
                                      sem_s.at[sg]).wait()

            pltpu.async_copy(xn_hbm.at[src_buf.at[sl, row]], rows_v.at[sg],
                             sem_g.at[sg])
            pltpu.async_copy(norm_hbm.at[dst_buf.at[sl, row]], nd_v.at[sg],
                             sem_g.at[sg])

        return carry

    lax.fori_loop(0, NG_MAX + P, body, 0)

    def drain(k, carry):
        g = ng - P + k
        ss = lax.rem(g, P)
        e2 = lax.rem(lax.div(g, KC), 3)
        r2 = lax.rem(g, KC)
        pltpu.make_async_copy(rows_v.at[ss], accx_sh.at[dst_buf.at[e2, r2]],
                              sem_s.at[ss]).wait()
        pltpu.make_async_copy(nd_v.at[ss], s_sh.at[src_buf.at[e2, r2]],
                              sem_s.at[ss]).wait()
        return carry

    lax.fori_loop(0, P, drain, 0)
    plsc.subcore_barrier()
    pltpu.sync_copy(accx_sh.at[pl.ds(s * SLICE, SLICE)],
                    accx_out.at[c, pl.ds(s * SLICE, SLICE)])
    pltpu.sync_copy(s_sh.at[pl.ds(s * SLICE, SLICE)],
                    s_out.at[c, pl.ds(s * SLICE, SLICE)])


_main_call = pl.kernel(
    _main_body,
    out_type=(
        jax.ShapeDtypeStruct((2, NPAD, D), F32),
        jax.ShapeDtypeStruct((2, NPAD), F32),
    ),
    mesh=_mesh,
    compiler_params=_sc_params,
    scratch_types=[
        pltpu.VMEM((3, KC, 128), jnp.int32),
        pltpu.VMEM((3, KC, 128), jnp.int32),
        pltpu.VMEM((P, 128, D), F32),
        pltpu.VMEM((P, 128), F32),
        pltpu.VMEM_SHARED((NPAD, D), F32),
        pltpu.VMEM_SHARED((NPAD,), F32),
        pltpu.SemaphoreType.DMA((3,)),
        pltpu.SemaphoreType.DMA((P,)),
        pltpu.SemaphoreType.DMA((P,)),
    ],
)


def _tc1_body(d0_ref, d1_ref, x_ref, norm_ref, xn_ref):
    deg = d0_ref[...] + d1_ref[...] + 1.0
    nrm = lax.rsqrt(deg)
    norm_ref[...] = nrm
    xn_ref[...] = x_ref[...] * nrm


_GRID1 = 16


def _tc1(d0c, d1c, x_pad):
    blk = NPAD // _GRID1
    return pl.pallas_call(
        _tc1_body,
        grid=(_GRID1,),
        in_specs=[
            pl.BlockSpec((blk, 1), lambda i: (i, 0)),
            pl.BlockSpec((blk, 1), lambda i: (i, 0)),
            pl.BlockSpec((blk, D), lambda i: (i, 0)),
        ],
        out_specs=[
            pl.BlockSpec((blk, 1), lambda i: (i, 0)),
            pl.BlockSpec((blk, D), lambda i: (i, 0)),
        ],
        out_shape=[
            jax.ShapeDtypeStruct((NPAD, 1), F32),
            jax.ShapeDtypeStruct((NPAD, D), F32),
        ],
    )(d0c, d1c, x_pad)


def _tc2_body(a0_ref, a1_ref, xn_ref, s0_ref, s1_ref, norm_ref,
              w1_ref, b1_ref, w2_ref, b2_ref, wa_ref, ba_ref,
              out_ref, r_acc):
    k = pl.program_id(0)

    @pl.when(k == 0)
    def _():
        r_acc[...] = jnp.zeros_like(r_acc)

    blk = a0_ref.shape[0]
    nrm = norm_ref[...]
    u = a0_ref[...] + a1_ref[...] + xn_ref[...]
    h1 = jax.nn.relu(nrm * jnp.dot(u, w1_ref[...],
                                   preferred_element_type=F32) + b1_ref[...])
    wv = nrm * (nrm + s0_ref[...] + s1_ref[...])
    gidx = lax.broadcasted_iota(jnp.int32, (blk, 1), 0) + k * blk
    wv = jnp.where(gidx < N, wv, 0.0)
    r_acc[...] += jnp.sum(wv * h1, axis=0, keepdims=True)

    @pl.when(k == _GRID1 - 1)
    def _():
        feat = jnp.dot(r_acc[...], w2_ref[...],
                       preferred_element_type=F32) * (1.0 / N) + b2_ref[...]
        out_ref[...] = jnp.dot(feat, wa_ref[...],
                               preferred_element_type=F32) + ba_ref[...]


def _tc2(a0, a1, xn, s0c, s1c, norm_c, W1p, b1r, W2, b2r, Wa, bar):
    blk = NPAD // _GRID1
    wspec = lambda shape: pl.BlockSpec(shape, lambda i: (0, 0))
    return pl.pallas_call(
        _tc2_body,
        grid=(_GRID1,),
        in_specs=[
            pl.BlockSpec((blk, D), lambda i: (i, 0)),
            pl.BlockSpec((blk, D), lambda i: (i, 0)),
            pl.BlockSpec((blk, D), lambda i: (i, 0)),
            pl.BlockSpec((blk, 1), lambda i: (i, 0)),
            pl.BlockSpec((blk, 1), lambda i: (i, 0)),
            pl.BlockSpec((blk, 1), lambda i: (i, 0)),
            wspec((D, H)),
            wspec((1, H)),
            wspec((H, 64)),
            wspec((1, 64)),
            wspec((64, 10)),
            wspec((1, 10)),
        ],
        out_specs=pl.BlockSpec((1, 10), lambda i: (0, 0)),
        out_shape=jax.ShapeDtypeStruct((1, 10), F32),
        scratch_shapes=[pltpu.VMEM((1, H), F32)],
    )(a0, a1, xn, s0c, s1c, norm_c, W1p, b1r, W2, b2r, Wa, bar)


def kernel(x, edge_index, W1, b1, W2, b2, Wa, ba):
    edge_r = edge_index.reshape(2, NB, 128)
    x_pad = jnp.zeros((NPAD, D), F32).at[:N, :5].set(x)
    W1p = jnp.zeros((D, H), F32).at[:5, :].set(W1)
    ones128 = jnp.ones((128,), F32)
    zeros_s = jnp.zeros((SLICE,), F32)
    zeros_r = jnp.zeros((SLICE, D), F32)

    deg_out = _deg_call(edge_r, ones128, zeros_s)

    d0c = deg_out[0].reshape(NPAD, 1)
    d1c = deg_out[1].reshape(NPAD, 1)
    norm_c, xn = _tc1(d0c, d1c, x_pad)

    accx_out, s_out = _main_call(edge_r, xn, norm_c.reshape(NPAD), zeros_r,
                                 zeros_s)

    logits = _tc2(accx_out[0], accx_out[1], xn,
                  s_out[0].reshape(NPAD, 1), s_out[1].reshape(NPAD, 1),
                  norm_c, W1p, b1.reshape(1, H), W2, b2.reshape(1, 64),
                  Wa, ba.reshape(1, 10))
    return logits


# speedup vs baseline: 106.6743x; 1.2386x over previous
import jax
import jax.numpy as jnp
from jax import lax
from jax.experimental import pallas as pl
from jax.experimental.pallas import tpu as pltpu
from jax.experimental.pallas import tpu_sc as plsc

N = 100000
NPAD = 100352
SLICE = NPAD // 16
E = 3200000
NB = E // 128
KC = 8
NCH = NB // KC
CH_LO = NCH // 32
CH_EX = NCH % 32
NG_MAX = (CH_LO + 1) * KC
D = 8
H = 16
P = 4
F32 = jnp.float32

_mesh = plsc.VectorSubcoreMesh(core_axis_name="c", subcore_axis_name="s")
_sc_params = pltpu.CompilerParams(use_tc_tiling_on_sc=False)


def _worker_range(c, s):
    wid = c * 16 + s
    start = (wid * CH_LO + jnp.minimum(wid, CH_EX)) * KC
    ng = (CH_LO + jnp.where(wid < CH_EX, 1, 0)) * KC
    return start, ng


def _deg_body(edge_ref, ones_hbm, zs_hbm, deg_out,
              dst_buf, ones_vm, deg_sh, sem_e, sem_s):
    c = lax.axis_index("c")
    s = lax.axis_index("s")
    start, ng = _worker_range(c, s)
    pltpu.sync_copy(ones_hbm, ones_vm)
    pltpu.sync_copy(zs_hbm, deg_sh.at[pl.ds(s * SLICE, SLICE)])
    plsc.subcore_barrier()
    pltpu.async_copy(edge_ref.at[1, pl.ds(start, KC)], dst_buf.at[0],
                     sem_e.at[0])

    def body(t, carry):
        sl = lax.rem(lax.div(t, KC), 3)
        row = lax.rem(t, KC)

        @pl.when((row == 0) & (t + KC < ng))
        def _():
            so = lax.rem(lax.div(t, KC) + 1, 3)
            pltpu.async_copy(edge_ref.at[1, pl.ds(start + t + KC, KC)],
                             dst_buf.at[so], sem_e.at[so])

        @pl.when((row == 0) & (t < ng))
        def _():
            pltpu.make_async_copy(edge_ref.at[1, pl.ds(start + t, KC)],
                                  dst_buf.at[sl], sem_e.at[sl]).wait()

        @pl.when((t >= P) & (t - P < ng))
        def _():
            sp = lax.rem(t - P, P)
            sp_sl = lax.rem(lax.div(t - P, KC), 3)
            sp_row = lax.rem(t - P, KC)
            pltpu.make_async_copy(ones_vm,
                                  deg_sh.at[dst_buf.at[sp_sl, sp_row]],
                                  sem_s.at[sp]).wait()

        @pl.when(t < ng)
        def _():
            pltpu.async_copy(ones_vm, deg_sh.at[dst_buf.at[sl, row]],
                             sem_s.at[lax.rem(t, P)], add=True)

        return carry

    lax.fori_loop(0, NG_MAX + P, body, 0)
    plsc.subcore_barrier()
    pltpu.sync_copy(deg_sh.at[pl.ds(s * SLICE, SLICE)],
                    deg_out.at[c, pl.ds(s * SLICE, SLICE)])


_deg_call = pl.kernel(
    _deg_body,
    out_type=jax.ShapeDtypeStruct((2, NPAD), F32),
    mesh=_mesh,
    compiler_params=_sc_params,
    scratch_types=[
        pltpu.VMEM((3, KC, 128), jnp.int32),
        pltpu.VMEM((128,), F32),
        pltpu.VMEM_SHARED((NPAD,), F32),
        pltpu.SemaphoreType.DMA((3,)),
        pltpu.SemaphoreType.DMA((P,)),
    ],
)


def _main_body(edge_ref, xn_hbm, norm_hbm, zr_hbm, zs_hbm,
               accx_out, s_out,
               src_buf, dst_buf, rows_v, nd_v, accx_sh, s_sh,
               sem_e, sem_g, sem_s):
    c = lax.axis_index("c")
    s = lax.axis_index("s")
    start, ng = _worker_range(c, s)
    pltpu.sync_copy(zr_hbm, accx_sh.at[pl.ds(s * SLICE, SLICE)])
    pltpu.sync_copy(zs_hbm, s_sh.at[pl.ds(s * SLICE, SLICE)])
    plsc.subcore_barrier()
    pltpu.async_copy(edge_ref.at[0, pl.ds(start, KC)], src_buf.at[0],
                     sem_e.at[0])
    pltpu.async_copy(edge_ref.at[1, pl.ds(start, KC)], dst_buf.at[0],
                     sem_e.at[0])

    def body(t, carry):
        sl = lax.rem(lax.div(t, KC), 3)
        row = lax.rem(t, KC)

        @pl.when((row == 0) & (t + KC < ng))
        def _():
            so = lax.rem(lax.div(t, KC) + 1, 3)
            pltpu.async_copy(edge_ref.at[0, pl.ds(start + t + KC, KC)],
                             src_buf.at[so], sem_e.at[so])
            pltpu.async_copy(edge_ref.at[1, pl.ds(start + t + KC, KC)],
                             dst_buf.at[so], sem_e.at[so])

        @pl.when((row == 0) & (t < ng))
        def _():
            pltpu.make_async_copy(edge_ref.at[0, pl.ds(start + t, KC)],
                                  src_buf.at[sl], sem_e.at[sl]).wait()
            pltpu.make_async_copy(edge_ref.at[1, pl.ds(start + t, KC)],
                                  dst_buf.at[sl], sem_e.at[sl]).wait()

        @pl.when((t >= 2) & (t - 2 < ng))
        def _():
            g2 = t - 2
            ss = lax.rem(g2, P)
            e2 = lax.rem(lax.div(g2, KC), 3)
            r2 = lax.rem(g2, KC)
            pltpu.make_async_copy(xn_hbm.at[src_buf.at[e2, r2]],
                                  rows_v.at[ss], sem_g.at[ss]).wait()
            pltpu.make_async_copy(norm_hbm.at[dst_buf.at[e2, r2]],
                                  nd_v.at[ss], sem_g.at[ss]).wait()
            pltpu.async_copy(rows_v.at[ss], accx_sh.at[dst_buf.at[e2, r2]],
                             sem_s.at[ss], add=True)
            pltpu.async_copy(nd_v.at[ss], s_sh.at[src_buf.at[e2, r2]],
                             sem_s.at[ss], add=True)

        @pl.when(t < ng)
        def _():
            sg = lax.rem(t, P)

            @pl.when(t >= P)
            def _():
                gp = t - P
                ep = lax.rem(lax.div(gp, KC), 3)
                rp = lax.rem(gp, KC)
                pltpu.make_async_copy(rows_v.at[sg],
                                      accx_sh.at[dst_buf.at[ep, rp]],
                                      sem_s.at[sg]).wait()
                pltpu.make_async_copy(nd_v.at[sg],
                                      s_sh.at[src_buf.at[ep, rp]],
                                      sem_s.at[sg]).wait()

            pltpu.async_copy(xn_hbm.at[src_buf.at[sl, row]], rows_v.at[sg],
                             sem_g.at[sg])
            pltpu.async_copy(norm_hbm.at[dst_buf.at[sl, row]], nd_v.at[sg],
                             sem_g.at[sg])

        return carry

    lax.fori_loop(0, NG_MAX + P, body, 0)

    def drain(k, carry):
        g = ng - P + k
        ss = lax.rem(g, P)
        e2 = lax.rem(lax.div(g, KC), 3)
        r2 = lax.rem(g, KC)
        pltpu.make_async_copy(rows_v.at[ss], accx_sh.at[dst_buf.at[e2, r2]],
                              sem_s.at[ss]).wait()
        pltpu.make_async_copy(nd_v.at[ss], s_sh.at[src_buf.at[e2, r2]],
                              sem_s.at[ss]).wait()
        return carry

    lax.fori_loop(0, P, drain, 0)
    plsc.subcore_barrier()
    pltpu.sync_copy(accx_sh.at[pl.ds(s * SLICE, SLICE)],
                    accx_out.at[c, pl.ds(s * SLICE, SLICE)])
    pltpu.sync_copy(s_sh.at[pl.ds(s * SLICE, SLICE)],
                    s_out.at[c, pl.ds(s * SLICE, SLICE)])


_main_call = pl.kernel(
    _main_body,
    out_type=(
        jax.ShapeDtypeStruct((2, NPAD, D), F32),
        jax.ShapeDtypeStruct((2, NPAD), F32),
    ),
    mesh=_mesh,
    compiler_params=_sc_params,
    scratch_types=[
        pltpu.VMEM((3, KC, 128), jnp.int32),
        pltpu.VMEM((3, KC, 128), jnp.int32),
        pltpu.VMEM((P, 128, D), F32),
        pltpu.VMEM((P, 128), F32),
        pltpu.VMEM_SHARED((NPAD, D), F32),
        pltpu.VMEM_SHARED((NPAD,), F32),
        pltpu.SemaphoreType.DMA((3,)),
        pltpu.SemaphoreType.DMA((P,)),
        pltpu.SemaphoreType.DMA((P,)),
    ],
)


def _tc1_body(d0v_ref, d1v_ref, d0f_ref, d1f_ref, xf_ref, r8_ref,
              xnf_ref, n16_ref, nf_ref):
    deg16 = d0v_ref[...] + d1v_ref[...] + 1.0
    nrm16 = lax.rsqrt(deg16)
    n16_ref[...] = nrm16
    nf_ref[...] = lax.rsqrt(d0f_ref[...] + d1f_ref[...] + 1.0)
    nrep8 = jnp.dot(nrm16, r8_ref[...], preferred_element_type=F32)
    xnf_ref[...] = xf_ref[...] * nrep8


_GRID = 7
_RF = NPAD * D // 128
_RN = NPAD // 128


def _tc1(d0v, d1v, d0f, d1f, xflat, R8):
    bf = _RF // _GRID
    bn = _RN // _GRID
    return pl.pallas_call(
        _tc1_body,
        grid=(_GRID,),
        in_specs=[
            pl.BlockSpec((bf, 16), lambda i: (i, 0)),
            pl.BlockSpec((bf, 16), lambda i: (i, 0)),
            pl.BlockSpec((bn, 128), lambda i: (i, 0)),
            pl.BlockSpec((bn, 128), lambda i: (i, 0)),
            pl.BlockSpec((bf, 128), lambda i: (i, 0)),
            pl.BlockSpec((16, 128), lambda i: (0, 0)),
        ],
        out_specs=[
            pl.BlockSpec((bf, 128), lambda i: (i, 0)),
            pl.BlockSpec((bf, 16), lambda i: (i, 0)),
            pl.BlockSpec((bn, 128), lambda i: (i, 0)),
        ],
        out_shape=[
            jax.ShapeDtypeStruct((_RF, 128), F32),
            jax.ShapeDtypeStruct((_RF, 16), F32),
            jax.ShapeDtypeStruct((_RN, 128), F32),
        ],
    )(d0v, d1v, d0f, d1f, xflat, R8)


def _tc2_body(a0_ref, a1_ref, xn_ref, s0_ref, s1_ref, n16_ref,
              w1b_ref, b1r_ref, r16_ref, fold_ref, w2_ref, b2_ref,
              wa_ref, ba_ref, out_ref, r_acc):
    k = pl.program_id(0)

    @pl.when(k == 0)
    def _():
        r_acc[...] = jnp.zeros_like(r_acc)

    blk = a0_ref.shape[0]
    u = a0_ref[...] + a1_ref[...] + xn_ref[...]
    pre = jnp.dot(u, w1b_ref[...], preferred_element_type=F32)
    nrm16 = n16_ref[...]
    nrep16 = jnp.dot(nrm16, r16_ref[...], preferred_element_type=F32)
    h1 = jax.nn.relu(nrep16 * pre + b1r_ref[...])
    wv = nrm16 * (nrm16 + s0_ref[...] + s1_ref[...])
    gidx = ((lax.broadcasted_iota(jnp.int32, (blk, 16), 0) + k * blk) * 16
            + lax.broadcasted_iota(jnp.int32, (blk, 16), 1))
    wv = jnp.where(gidx < N, wv, 0.0)
    wrep = jnp.dot(wv, r16_ref[...], preferred_element_type=F32)
    r_acc[...] += jnp.sum(wrep * h1, axis=0, keepdims=True)

    @pl.when(k == _GRID - 1)
    def _():
        r16 = jnp.dot(r_acc[...], fold_ref[...], preferred_element_type=F32)
        feat = jnp.dot(r16, w2_ref[...],
                       preferred_element_type=F32) * (1.0 / N) + b2_ref[...]
        out_ref[...] = jnp.dot(feat, wa_ref[...],
                               preferred_element_type=F32) + ba_ref[...]


def _tc2(a0f, a1f, xnf, s0v, s1v, n16, W1big, b1rep, R16, F16,
         W2, b2r, Wa, bar):
    bf = _RF // _GRID
    wspec = lambda shape: pl.BlockSpec(shape, lambda i: (0, 0))
    return pl.pallas_call(
        _tc2_body,
        grid=(_GRID,),
        in_specs=[
            pl.BlockSpec((bf, 128), lambda i: (i, 0)),
            pl.BlockSpec((bf, 128), lambda i: (i, 0)),
            pl.BlockSpec((bf, 128), lambda i: (i, 0)),
            pl.BlockSpec((bf, 16), lambda i: (i, 0)),
            pl.BlockSpec((bf, 16), lambda i: (i, 0)),
            pl.BlockSpec((bf, 16), lambda i: (i, 0)),
            wspec((128, 256)),
            wspec((1, 256)),
            wspec((16, 256)),
            wspec((256, 16)),
            wspec((H, 64)),
            wspec((1, 64)),
            wspec((64, 10)),
            wspec((1, 10)),
        ],
        out_specs=pl.BlockSpec((1, 10), lambda i: (0, 0)),
        out_shape=jax.ShapeDtypeStruct((1, 10), F32),
        scratch_shapes=[pltpu.VMEM((1, 256), F32)],
    )(a0f, a1f, xnf, s0v, s1v, n16, W1big, b1rep, R16, F16,
      W2, b2r, Wa, bar)


def kernel(x, edge_index, W1, b1, W2, b2, Wa, ba):
    edge_r = edge_index.reshape(2, NB, 128)
    xflat = jnp.pad(x, ((0, NPAD - N), (0, D - 5))).reshape(_RF, 128)
    W1p = jnp.zeros((D, H), F32).at[:5, :].set(W1)
    eye16 = jnp.eye(16, dtype=F32)
    W1big = jnp.kron(eye16, W1p)
    R8 = jnp.kron(eye16, jnp.ones((1, D), F32))
    R16 = jnp.kron(eye16, jnp.ones((1, H), F32))
    F16 = jnp.tile(eye16, (16, 1))
    b1rep = jnp.tile(b1, 16).reshape(1, 256)
    ones128 = jnp.ones((128,), F32)
    zeros_s = jnp.zeros((SLICE,), F32)
    zeros_r = jnp.zeros((SLICE, D), F32)

    deg_out = _deg_call(edge_r, ones128, zeros_s)

    d0v = deg_out[0].reshape(_RF, 16)
    d1v = deg_out[1].reshape(_RF, 16)
    d0f = deg_out[0].reshape(_RN, 128)
    d1f = deg_out[1].reshape(_RN, 128)
    xnf, n16, normf = _tc1(d0v, d1v, d0f, d1f, xflat, R8)

    accx_out, s_out = _main_call(edge_r, xnf.reshape(NPAD, D),
                                 normf.reshape(NPAD), zeros_r, zeros_s)

    logits = _tc2(accx_out[0].reshape(_RF, 128), accx_out[1].reshape(_RF, 128),
                  xnf, s_out[0].reshape(_RF, 16), s_out[1].reshape(_RF, 16),
                  n16, W1big, b1rep, R16, F16,
                  W2, b2.reshape(1, 64), Wa, ba.reshape(1, 10))
    return logits


# speedup vs baseline: 144.1227x; 1.3511x over previous
import jax
import jax.numpy as jnp
from jax import lax
from jax.experimental import pallas as pl
from jax.experimental.pallas import tpu as pltpu
from jax.experimental.pallas import tpu_sc as plsc

N = 100000
NPAD = 100352
SLICE = NPAD // 16
E = 3200000
NB = E // 128
KC = 8
NCH = NB // KC
CH_LO = NCH // 32
CH_EX = NCH % 32
NG_MAX = (CH_LO + 1) * KC
D = 8
H = 16
P = 4
F32 = jnp.float32

_mesh = plsc.VectorSubcoreMesh(core_axis_name="c", subcore_axis_name="s")
_sc_params = pltpu.CompilerParams(use_tc_tiling_on_sc=False)


def _worker_range(c, s):
    wid = c * 16 + s
    start = (wid * CH_LO + jnp.minimum(wid, CH_EX)) * KC
    ng = (CH_LO + jnp.where(wid < CH_EX, 1, 0)) * KC
    return start, ng


def _deg_body(edge_ref, ones_hbm, zs_hbm, deg0_out, deg1_out,
              dst_buf, ones_vm, deg_sh, sem_e, sem_s):
    c = lax.axis_index("c")
    s = lax.axis_index("s")
    start, ng = _worker_range(c, s)
    pltpu.sync_copy(ones_hbm, ones_vm)
    pltpu.sync_copy(zs_hbm, deg_sh.at[pl.ds(s * SLICE, SLICE)])
    plsc.subcore_barrier()
    pltpu.async_copy(edge_ref.at[1, pl.ds(start, KC)], dst_buf.at[0],
                     sem_e.at[0])

    def body(t, carry):
        sl = lax.rem(lax.div(t, KC), 3)
        row = lax.rem(t, KC)

        @pl.when((row == 0) & (t + KC < ng))
        def _():
            so = lax.rem(lax.div(t, KC) + 1, 3)
            pltpu.async_copy(edge_ref.at[1, pl.ds(start + t + KC, KC)],
                             dst_buf.at[so], sem_e.at[so])

        @pl.when((row == 0) & (t < ng))
        def _():
            pltpu.make_async_copy(edge_ref.at[1, pl.ds(start + t, KC)],
                                  dst_buf.at[sl], sem_e.at[sl]).wait()

        @pl.when((t >= P) & (t - P < ng))
        def _():
            sp = lax.rem(t - P, P)
            sp_sl = lax.rem(lax.div(t - P, KC), 3)
            sp_row = lax.rem(t - P, KC)
            pltpu.make_async_copy(ones_vm,
                                  deg_sh.at[dst_buf.at[sp_sl, sp_row]],
                                  sem_s.at[sp]).wait()

        @pl.when(t < ng)
        def _():
            pltpu.async_copy(ones_vm, deg_sh.at[dst_buf.at[sl, row]],
                             sem_s.at[lax.rem(t, P)], add=True)

        return carry

    lax.fori_loop(0, NG_MAX + P, body, 0)
    plsc.subcore_barrier()

    @pl.when(c == 0)
    def _():
        pltpu.sync_copy(deg_sh.at[pl.ds(s * SLICE, SLICE)],
                        deg0_out.at[pl.ds(s * SLICE, SLICE)])

    @pl.when(c == 1)
    def _():
        pltpu.sync_copy(deg_sh.at[pl.ds(s * SLICE, SLICE)],
                        deg1_out.at[pl.ds(s * SLICE, SLICE)])


_deg_call = pl.kernel(
    _deg_body,
    out_type=(jax.ShapeDtypeStruct((NPAD,), F32),
              jax.ShapeDtypeStruct((NPAD,), F32)),
    mesh=_mesh,
    compiler_params=_sc_params,
    scratch_types=[
        pltpu.VMEM((3, KC, 128), jnp.int32),
        pltpu.VMEM((128,), F32),
        pltpu.VMEM_SHARED((NPAD,), F32),
        pltpu.SemaphoreType.DMA((3,)),
        pltpu.SemaphoreType.DMA((P,)),
    ],
)


def _main_body(edge_ref, xn_hbm, norm_hbm, zr_hbm, zs_hbm,
               accx0_out, accx1_out, s0_out, s1_out,
               src_buf, dst_buf, rows_v, nd_v, accx_sh, s_sh,
               sem_e, sem_g, sem_s):
    c = lax.axis_index("c")
    s = lax.axis_index("s")
    start, ng = _worker_range(c, s)
    pltpu.sync_copy(zr_hbm, accx_sh.at[pl.ds(s * SLICE, SLICE)])
    pltpu.sync_copy(zs_hbm, s_sh.at[pl.ds(s * SLICE, SLICE)])
    plsc.subcore_barrier()
    pltpu.async_copy(edge_ref.at[0, pl.ds(start, KC)], src_buf.at[0],
                     sem_e.at[0])
    pltpu.async_copy(edge_ref.at[1, pl.ds(start, KC)], dst_buf.at[0],
                     sem_e.at[0])

    def body(t, carry):
        sl = lax.rem(lax.div(t, KC), 3)
        row = lax.rem(t, KC)

        @pl.when((row == 0) & (t + KC < ng))
        def _():
            so = lax.rem(lax.div(t, KC) + 1, 3)
            pltpu.async_copy(edge_ref.at[0, pl.ds(start + t + KC, KC)],
                             src_buf.at[so], sem_e.at[so])
            pltpu.async_copy(edge_ref.at[1, pl.ds(start + t + KC, KC)],
                             dst_buf.at[so], sem_e.at[so])

        @pl.when((row == 0) & (t < ng))
        def _():
            pltpu.make_async_copy(edge_ref.at[0, pl.ds(start + t, KC)],
                                  src_buf.at[sl], sem_e.at[sl]).wait()
            pltpu.make_async_copy(edge_ref.at[1, pl.ds(start + t, KC)],
                                  dst_buf.at[sl], sem_e.at[sl]).wait()

        @pl.when((t >= 2) & (t - 2 < ng))
        def _():
            g2 = t - 2
            ss = lax.rem(g2, P)
            e2 = lax.rem(lax.div(g2, KC), 3)
            r2 = lax.rem(g2, KC)
            pltpu.make_async_copy(xn_hbm.at[src_buf.at[e2, r2]],
                                  rows_v.at[ss], sem_g.at[ss]).wait()
            pltpu.make_async_copy(norm_hbm.at[dst_buf.at[e2, r2]],
                                  nd_v.at[ss], sem_g.at[ss]).wait()
            pltpu.async_copy(rows_v.at[ss], accx_sh.at[dst_buf.at[e2, r2]],
                             sem_s.at[ss], add=True)
            pltpu.async_copy(nd_v.at[ss], s_sh.at[src_buf.at[e2, r2]],
                             sem_s.at[ss], add=True)

        @pl.when(t < ng)
        def _():
            sg = lax.rem(t, P)

            @pl.when(t >= P)
            def _():
                gp = t - P
                ep = lax.rem(lax.div(gp, KC), 3)
                rp = lax.rem(gp, KC)
                pltpu.make_async_copy(rows_v.at[sg],
                                      accx_sh.at[dst_buf.at[ep, rp]],
                                      sem_s.at[sg]).wait()
                pltpu.make_async_copy(nd_v.at[sg],
                                      s_sh.at[src_buf.at[ep, rp]],
                                      sem_s.at[sg]).wait()

            pltpu.async_copy(xn_hbm.at[src_buf.at[sl, row]], rows_v.at[sg],
                             sem_g.at[sg])
            pltpu.async_copy(norm_hbm.at[dst_buf.at[sl, row]], nd_v.at[sg],
                             sem_g.at[sg])

        return carry

    lax.fori_loop(0, NG_MAX + P, body, 0)

    def drain(k, carry):
        g = ng - P + k
        ss = lax.rem(g, P)
        e2 = lax.rem(lax.div(g, KC), 3)
        r2 = lax.rem(g, KC)
        pltpu.make_async_copy(rows_v.at[ss], accx_sh.at[dst_buf.at[e2, r2]],
                              sem_s.at[ss]).wait()
        pltpu.make_async_copy(nd_v.at[ss], s_sh.at[src_buf.at[e2, r2]],
                              sem_s.at[ss]).wait()
        return carry

    lax.fori_loop(0, P, drain, 0)
    plsc.subcore_barrier()

    @pl.when(c == 0)
    def _():
        pltpu.sync_copy(accx_sh.at[pl.ds(s * SLICE, SLICE)],
                        accx0_out.at[pl.ds(s * SLICE, SLICE)])
        pltpu.sync_copy(s_sh.at[pl.ds(s * SLICE, SLICE)],
                        s0_out.at[pl.ds(s * SLICE, SLICE)])

    @pl.when(c == 1)
    def _():
        pltpu.sync_copy(accx_sh.at[pl.ds(s * SLICE, SLICE)],
                        accx1_out.at[pl.ds(s * SLICE, SLICE)])
        pltpu.sync_copy(s_sh.at[pl.ds(s * SLICE, SLICE)],
                        s1_out.at[pl.ds(s * SLICE, SLICE)])


_main_call = pl.kernel(
    _main_body,
    out_type=(
        jax.ShapeDtypeStruct((NPAD, D), F32),
        jax.ShapeDtypeStruct((NPAD, D), F32),
        jax.ShapeDtypeStruct((NPAD,), F32),
        jax.ShapeDtypeStruct((NPAD,), F32),
    ),
    mesh=_mesh,
    compiler_params=_sc_params,
    scratch_types=[
        pltpu.VMEM((3, KC, 128), jnp.int32),
        pltpu.VMEM((3, KC, 128), jnp.int32),
        pltpu.VMEM((P, 128, D), F32),
        pltpu.VMEM((P, 128), F32),
        pltpu.VMEM_SHARED((NPAD, D), F32),
        pltpu.VMEM_SHARED((NPAD,), F32),
        pltpu.SemaphoreType.DMA((3,)),
        pltpu.SemaphoreType.DMA((P,)),
        pltpu.SemaphoreType.DMA((P,)),
    ],
)


def _tc1_body(d0v_ref, d1v_ref, d0f_ref, d1f_ref, xf_ref, r8_ref,
              xnf_ref, n16_ref, nf_ref):
    deg16 = d0v_ref[...] + d1v_ref[...] + 1.0
    nrm16 = lax.rsqrt(deg16)
    n16_ref[...] = nrm16
    nf_ref[...] = lax.rsqrt(d0f_ref[...] + d1f_ref[...] + 1.0)
    nrep8 = jnp.dot(nrm16, r8_ref[...], preferred_element_type=F32)
    xnf_ref[...] = xf_ref[...] * nrep8


_GRID = 7
_RF = NPAD * D // 128
_RN = NPAD // 128


def _tc1(d0v, d1v, d0f, d1f, xflat, R8):
    bf = _RF // _GRID
    bn = _RN // _GRID
    return pl.pallas_call(
        _tc1_body,
        grid=(_GRID,),
        in_specs=[
            pl.BlockSpec((bf, 16), lambda i: (i, 0)),
            pl.BlockSpec((bf, 16), lambda i: (i, 0)),
            pl.BlockSpec((bn, 128), lambda i: (i, 0)),
            pl.BlockSpec((bn, 128), lambda i: (i, 0)),
            pl.BlockSpec((bf, 128), lambda i: (i, 0)),
            pl.BlockSpec((16, 128), lambda i: (0, 0)),
        ],
        out_specs=[
            pl.BlockSpec((bf, 128), lambda i: (i, 0)),
            pl.BlockSpec((bf, 16), lambda i: (i, 0)),
            pl.BlockSpec((bn, 128), lambda i: (i, 0)),
        ],
        out_shape=[
            jax.ShapeDtypeStruct((_RF, 128), F32),
            jax.ShapeDtypeStruct((_RF, 16), F32),
            jax.ShapeDtypeStruct((_RN, 128), F32),
        ],
    )(d0v, d1v, d0f, d1f, xflat, R8)


def _tc2_body(a0_ref, a1_ref, xn_ref, s0_ref, s1_ref, n16_ref,
              w1b_ref, b1r_ref, r16_ref, fold_ref, w2_ref, b2_ref,
              wa_ref, ba_ref, out_ref, r_acc):
    k = pl.program_id(0)

    @pl.when(k == 0)
    def _():
        r_acc[...] = jnp.zeros_like(r_acc)

    blk = a0_ref.shape[0]
    u = a0_ref[...] + a1_ref[...] + xn_ref[...]
    pre = jnp.dot(u, w1b_ref[...], preferred_element_type=F32)
    nrm16 = n16_ref[...]
    nrep16 = jnp.dot(nrm16, r16_ref[...], preferred_element_type=F32)
    h1 = jax.nn.relu(nrep16 * pre + b1r_ref[...])
    wv = nrm16 * (nrm16 + s0_ref[...] + s1_ref[...])
    gidx = ((lax.broadcasted_iota(jnp.int32, (blk, 16), 0) + k * blk) * 16
            + lax.broadcasted_iota(jnp.int32, (blk, 16), 1))
    wv = jnp.where(gidx < N, wv, 0.0)
    wrep = jnp.dot(wv, r16_ref[...], preferred_element_type=F32)
    r_acc[...] += jnp.sum(wrep * h1, axis=0, keepdims=True)

    @pl.when(k == _GRID - 1)
    def _():
        r16 = jnp.dot(r_acc[...], fold_ref[...], preferred_element_type=F32)
        feat = jnp.dot(r16, w2_ref[...],
                       preferred_element_type=F32) * (1.0 / N) + b2_ref[...]
        out_ref[...] = jnp.dot(feat, wa_ref[...],
                               preferred_element_type=F32) + ba_ref[...]


def _tc2(a0f, a1f, xnf, s0v, s1v, n16, W1big, b1rep, R16, F16,
         W2, b2r, Wa, bar):
    bf = _RF // _GRID
    wspec = lambda shape: pl.BlockSpec(shape, lambda i: (0, 0))
    return pl.pallas_call(
        _tc2_body,
        grid=(_GRID,),
        in_specs=[
            pl.BlockSpec((bf, 128), lambda i: (i, 0)),
            pl.BlockSpec((bf, 128), lambda i: (i, 0)),
            pl.BlockSpec((bf, 128), lambda i: (i, 0)),
            pl.BlockSpec((bf, 16), lambda i: (i, 0)),
            pl.BlockSpec((bf, 16), lambda i: (i, 0)),
            pl.BlockSpec((bf, 16), lambda i: (i, 0)),
            wspec((128, 256)),
            wspec((1, 256)),
            wspec((16, 256)),
            wspec((256, 16)),
            wspec((H, 64)),
            wspec((1, 64)),
            wspec((64, 10)),
            wspec((1, 10)),
        ],
        out_specs=pl.BlockSpec((1, 10), lambda i: (0, 0)),
        out_shape=jax.ShapeDtypeStruct((1, 10), F32),
        scratch_shapes=[pltpu.VMEM((1, 256), F32)],
    )(a0f, a1f, xnf, s0v, s1v, n16, W1big, b1rep, R16, F16,
      W2, b2r, Wa, bar)


def kernel(x, edge_index, W1, b1, W2, b2, Wa, ba):
    edge_r = edge_index.reshape(2, NB, 128)
    xflat = jnp.pad(x, ((0, NPAD - N), (0, D - 5))).reshape(_RF, 128)
    W1p = jnp.zeros((D, H), F32).at[:5, :].set(W1)
    eye16 = jnp.eye(16, dtype=F32)
    W1big = jnp.kron(eye16, W1p)
    R8 = jnp.kron(eye16, jnp.ones((1, D), F32))
    R16 = jnp.kron(eye16, jnp.ones((1, H), F32))
    F16 = jnp.tile(eye16, (16, 1))
    b1rep = jnp.tile(b1, 16).reshape(1, 256)
    ones128 = jnp.ones((128,), F32)
    zeros_s = jnp.zeros((SLICE,), F32)
    zeros_r = jnp.zeros((SLICE, D), F32)

    deg0, deg1 = _deg_call(edge_r, ones128, zeros_s)

    d0v = deg0.reshape(_RF, 16)
    d1v = deg1.reshape(_RF, 16)
    d0f = deg0.reshape(_RN, 128)
    d1f = deg1.reshape(_RN, 128)
    xnf, n16, normf = _tc1(d0v, d1v, d0f, d1f, xflat, R8)

    accx0, accx1, s0, s1 = _main_call(edge_r, xnf.reshape(NPAD, D),
                                      normf.reshape(NPAD), zeros_r, zeros_s)

    logits = _tc2(accx0.reshape(_RF, 128), accx1.reshape(_RF, 128),
                  xnf, s0.reshape(_RF, 16), s1.reshape(_RF, 16),
                  n16, W1big, b1rep, R16, F16,
                  W2, b2.reshape(1, 64), Wa, ba.reshape(1, 10))
    return logits
